# baseline (speedup 1.0000x reference)
import jax
import jax.numpy as jnp
from jax import lax
from jax.experimental import pallas as pl
from jax.experimental.pallas import tpu as pltpu
from jax.experimental.pallas import tpu_sc as plsc

NUM_FIELDS = 26
VOCAB = 10000
D_EMBED = 100
BATCH = 16384
D_PAD = 112

_INFO = plsc.get_sparse_core_info()
NC = _INFO.num_cores
NS = _INFO.num_subcores
NW = NC * NS
L = _INFO.num_lanes

N_ROWS = BATCH * NUM_FIELDS
ROWS_PER_W = N_ROWS // NW
CHUNK = 128
NCHUNKS = ROWS_PER_W // CHUNK
NBUF = 2


def _gather_body(idx_hbm, tab_hbm, out_hbm, idx_all, rows_v, gsem, wsem):
    wid = lax.axis_index("s") * NC + lax.axis_index("c")
    wbase = wid * ROWS_PER_W
    pltpu.sync_copy(idx_hbm.at[pl.ds(wbase, ROWS_PER_W)], idx_all)

    def out_slice(c):
        return out_hbm.at[pl.ds(wbase + c * CHUNK, CHUNK)]

    def pair_body(i, carry):
        c0 = i * NBUF

        @pl.when(i > 0)
        def _():
            for b in range(NBUF):
                pltpu.make_async_copy(rows_v.at[b], out_slice(0), wsem[b]).wait()

        for b in range(NBUF):
            pltpu.async_copy(
                tab_hbm.at[idx_all.at[pl.ds((c0 + b) * CHUNK, CHUNK)]],
                rows_v.at[b], gsem[b])
        for b in range(NBUF):
            pltpu.make_async_copy(
                tab_hbm.at[idx_all.at[pl.ds((c0 + b) * CHUNK, CHUNK)]],
                rows_v.at[b], gsem[b]).wait()
            pltpu.async_copy(rows_v.at[b], out_slice(c0 + b), wsem[b])
        return carry

    lax.fori_loop(0, NCHUNKS // NBUF, pair_body, 0)
    for b in range(NBUF):
        pltpu.make_async_copy(rows_v.at[b], out_slice(0), wsem[b]).wait()


@jax.jit
def _gather(x_flat, flat_table):
    mesh = plsc.VectorSubcoreMesh(core_axis_name="c", subcore_axis_name="s")
    call = pl.kernel(
        _gather_body,
        out_type=jax.ShapeDtypeStruct((N_ROWS, D_PAD), jnp.float32),
        mesh=mesh,
        scratch_types=[
            pltpu.VMEM((ROWS_PER_W,), jnp.int32),
            pltpu.VMEM((NBUF, CHUNK, D_PAD), jnp.float32),
            [pltpu.SemaphoreType.DMA] * NBUF,
            [pltpu.SemaphoreType.DMA] * NBUF,
        ],
        compiler_params=pltpu.CompilerParams(
            use_tc_tiling_on_sc=False, needs_layout_passes=False),
    )
    return call(x_flat, flat_table)


def kernel(x_cat, tables):
    x_flat = (x_cat + jnp.arange(NUM_FIELDS, dtype=jnp.int32) * VOCAB).reshape(N_ROWS)
    flat_table = jnp.concatenate(
        [tables.reshape(NUM_FIELDS * VOCAB, D_EMBED),
         jnp.zeros((NUM_FIELDS * VOCAB, D_PAD - D_EMBED), jnp.float32)], axis=1)
    out = _gather(x_flat, flat_table)
    return out[:, :D_EMBED].reshape(BATCH, NUM_FIELDS * D_EMBED)


# speedup vs baseline: 2.0126x; 2.0126x over previous
import jax
import jax.numpy as jnp
from jax import lax
from jax.experimental import pallas as pl
from jax.experimental.pallas import tpu as pltpu
from jax.experimental.pallas import tpu_sc as plsc

NUM_FIELDS = 26
VOCAB = 10000
D_EMBED = 100
BATCH = 16384
D_PAD = 128
F_HALF = NUM_FIELDS // 2

_INFO = plsc.get_sparse_core_info()
NC = _INFO.num_cores
NS = _INFO.num_subcores
NW = NC * NS
L = _INFO.num_lanes

N_ROWS = BATCH * NUM_FIELDS
ROWS_PER_W = N_ROWS // NW
CHUNK = 128
NCHUNKS = ROWS_PER_W // CHUNK
NBUF = 2


def _gather_body(idx_hbm, taba_hbm, tabb_hbm, out_hbm, idx_all, rows_v,
                 gsem, wsem):
    wid = lax.axis_index("s") * NC + lax.axis_index("c")
    wbase = wid * ROWS_PER_W
    pltpu.sync_copy(idx_hbm.at[pl.ds(wbase, ROWS_PER_W)], idx_all)

    def out_slice(c):
        return out_hbm.at[pl.ds(wbase + c * CHUNK, CHUNK)]

    def run(tab_hbm):
        def pair_body(i, carry):
            c0 = i * NBUF

            @pl.when(i > 0)
            def _():
                for b in range(NBUF):
                    pltpu.make_async_copy(
                        rows_v.at[b], out_slice(0), wsem[b]).wait()

            for b in range(NBUF):
                pltpu.async_copy(
                    tab_hbm.at[idx_all.at[pl.ds((c0 + b) * CHUNK, CHUNK)]],
                    rows_v.at[b], gsem[b])
            for b in range(NBUF):
                pltpu.make_async_copy(
                    tab_hbm.at[idx_all.at[pl.ds((c0 + b) * CHUNK, CHUNK)]],
                    rows_v.at[b], gsem[b]).wait()
                pltpu.async_copy(rows_v.at[b], out_slice(c0 + b), wsem[b])
            return carry

        lax.fori_loop(0, NCHUNKS // NBUF, pair_body, 0)
        for b in range(NBUF):
            pltpu.make_async_copy(rows_v.at[b], out_slice(0), wsem[b]).wait()

    @pl.when(wbase < F_HALF * BATCH)
    def _():
        run(taba_hbm)

    @pl.when(wbase >= F_HALF * BATCH)
    def _():
        run(tabb_hbm)


@jax.jit
def _gather(x_flat, tab_a, tab_b):
    mesh = plsc.VectorSubcoreMesh(core_axis_name="c", subcore_axis_name="s")
    call = pl.kernel(
        _gather_body,
        out_type=jax.ShapeDtypeStruct((N_ROWS, D_PAD), jnp.float32),
        mesh=mesh,
        scratch_types=[
            pltpu.VMEM((ROWS_PER_W,), jnp.int32),
            pltpu.VMEM((NBUF, CHUNK, D_PAD), jnp.float32),
            [pltpu.SemaphoreType.DMA] * NBUF,
            [pltpu.SemaphoreType.DMA] * NBUF,
        ],
        compiler_params=pltpu.CompilerParams(
            use_tc_tiling_on_sc=False, needs_layout_passes=False),
    )
    return call(x_flat, tab_a, tab_b)


def _padded_half(tab_half):
    return lax.pad(
        tab_half, jnp.float32(0), ((0, 0, 0), (0, 0, 0), (0, D_PAD - D_EMBED, 0))
    ).reshape(F_HALF * VOCAB, D_PAD)


def kernel(x_cat, tables):
    f_ids = jnp.arange(NUM_FIELDS, dtype=jnp.int32)
    half_off = jnp.where(f_ids < F_HALF, f_ids, f_ids - F_HALF) * VOCAB
    x_flat = (x_cat.T + half_off[:, None]).reshape(N_ROWS)
    tab_a = _padded_half(tables[:F_HALF])
    tab_b = _padded_half(tables[F_HALF:])
    out = _gather(x_flat, tab_a, tab_b)
    out = out[:, :D_EMBED].reshape(NUM_FIELDS, BATCH, D_EMBED)
    return out.transpose(1, 0, 2).reshape(BATCH, NUM_FIELDS * D_EMBED)
